# baseline (speedup 1.0000x reference)
import functools

import jax
import jax.numpy as jnp
from jax import lax
from jax.experimental import pallas as pl
from jax.experimental.pallas import tpu as pltpu
from jax.experimental.pallas import tpu_sc as plsc


def _mesh_info():
    info = plsc.get_sparse_core_info()
    return info, plsc.VectorSubcoreMesh(core_axis_name="c", subcore_axis_name="s")


def _make_gather0(B, V, D, b_per_w, info, mesh):
    @functools.partial(
        pl.kernel,
        out_type=jax.ShapeDtypeStruct((B, D), jnp.float32),
        mesh=mesh,
        compiler_params=pltpu.CompilerParams(use_tc_tiling_on_sc=False),
        scratch_types=[
            pltpu.VMEM((b_per_w,), jnp.int32),
            pltpu.VMEM((b_per_w, D), jnp.float32),
            pltpu.SemaphoreType.DMA,
        ],
    )
    def run(idx_hbm, w_hbm, out_hbm, idx_v, rows_v, sem):
        wid = lax.axis_index("s") * info.num_cores + lax.axis_index("c")
        base = wid * b_per_w
        pltpu.sync_copy(idx_hbm.at[pl.ds(base, b_per_w)], idx_v)
        pltpu.async_copy(w_hbm.at[idx_v], rows_v, sem).wait()
        pltpu.sync_copy(rows_v, out_hbm.at[pl.ds(base, b_per_w)])

    return run


def _make_gather_add(B, V, D, b_per_w, info, mesh):
    @functools.partial(
        pl.kernel,
        out_type=jax.ShapeDtypeStruct((B, D), jnp.float32),
        mesh=mesh,
        compiler_params=pltpu.CompilerParams(use_tc_tiling_on_sc=False),
        scratch_types=[
            pltpu.VMEM((b_per_w,), jnp.int32),
            pltpu.VMEM((b_per_w, D), jnp.float32),
            pltpu.VMEM((b_per_w, D), jnp.float32),
            pltpu.SemaphoreType.DMA,
        ],
    )
    def run(idx_hbm, w_hbm, part_hbm, out_hbm, idx_v, rows_v, part_v, sem):
        wid = lax.axis_index("s") * info.num_cores + lax.axis_index("c")
        base = wid * b_per_w
        pltpu.sync_copy(idx_hbm.at[pl.ds(base, b_per_w)], idx_v)
        cp = pltpu.async_copy(w_hbm.at[idx_v], rows_v, sem)
        pltpu.sync_copy(part_hbm.at[pl.ds(base, b_per_w)], part_v)
        cp.wait()

        def add_rows(i, carry):
            for j in range(D // 16):
                sl = pl.ds(j * 16, 16)
                rows_v[i, sl] = rows_v[i, sl] + part_v[i, sl]
            return carry

        lax.fori_loop(0, b_per_w, add_rows, 0, unroll=8)
        pltpu.sync_copy(rows_v, out_hbm.at[pl.ds(base, b_per_w)])

    return run


def kernel(x, W0, W1):
    B = x.shape[0]
    V, D = W0.shape
    info, mesh = _mesh_info()
    NW = info.num_cores * info.num_subcores
    b_per_w = B // NW
    idx0 = x[:, 0].astype(jnp.int32)
    idx1 = x[:, 1].astype(jnp.int32)
    part = _make_gather0(B, V, D, b_per_w, info, mesh)(idx0, W0)
    return _make_gather_add(B, V, D, b_per_w, info, mesh)(idx1, W1, part)
